# Initial kernel scaffold; baseline (speedup 1.0000x reference)
import jax
import jax.numpy as jnp
from jax.experimental import pallas as pl


def kernel(inputs, table):
    raise NotImplementedError("write your pallas kernel here")


# speedup vs baseline: 3.6210x; 3.6210x over previous
import functools

import jax
import jax.numpy as jnp
from jax import lax
from jax.experimental import pallas as pl
from jax.experimental.pallas import tpu as pltpu
from jax.experimental.pallas import tpu_sc as plsc

VOCAB = 1000
EMBED = 64
BATCH = 4096
FIELDS = 26
B_TOTAL = BATCH * FIELDS
NUM_WORKERS = 32
B_PER_W = B_TOTAL // NUM_WORKERS
CHUNK = 128
N_CHUNKS = B_PER_W // CHUNK
NBUF = 2


def _sc_embedding_gather(table, idx2d):
    mesh = plsc.VectorSubcoreMesh(core_axis_name="c", subcore_axis_name="s")

    @functools.partial(
        pl.kernel,
        mesh=mesh,
        out_type=jax.ShapeDtypeStruct((B_TOTAL, EMBED), jnp.float32),
        compiler_params=pltpu.CompilerParams(use_tc_tiling_on_sc=False),
        scratch_types=[
            pltpu.VMEM((N_CHUNKS, CHUNK), jnp.int32),
            pltpu.VMEM((NBUF, CHUNK, EMBED), jnp.float32),
            pltpu.SemaphoreType.DMA,
        ],
    )
    def k(table_hbm, idx_hbm, out_hbm, idx_v, rows_v, gsem):
        wid = lax.axis_index("s") * 2 + lax.axis_index("c")
        chunk0 = wid * N_CHUNKS

        pltpu.sync_copy(idx_hbm.at[wid], idx_v)

        for b in range(NBUF):
            pltpu.async_copy(table_hbm.at[idx_v.at[b]], rows_v.at[b], gsem)

        def body(g0, _):
            for b in range(NBUF):
                g = g0 + b
                pltpu.make_async_copy(
                    table_hbm.at[idx_v.at[g]], rows_v.at[b], gsem
                ).wait()
                pltpu.sync_copy(
                    rows_v.at[b],
                    out_hbm.at[pl.ds((chunk0 + g) * CHUNK, CHUNK)],
                )
                ng = g + NBUF

                @pl.when(ng < N_CHUNKS)
                def _():
                    pltpu.async_copy(
                        table_hbm.at[idx_v.at[ng]], rows_v.at[b], gsem
                    )

            return ()

        lax.fori_loop(0, N_CHUNKS // NBUF, lambda i, c: body(i * NBUF, c),
                      (), unroll=False)

    return k(table, idx2d)


def kernel(inputs, table):
    idx3d = inputs.reshape(NUM_WORKERS, N_CHUNKS, CHUNK)
    out = _sc_embedding_gather(table, idx3d)
    return out.reshape(BATCH, FIELDS, EMBED)
